# baseline (speedup 1.0000x reference)
import jax
import jax.numpy as jnp
from jax.experimental import pallas as pl
from jax.experimental.pallas import tpu as pltpu

_NROW = 100000
_D = 64
_CH = 12500
_NC = _NROW // _CH
_TOT = 2 * _NC


def _copy_body(u_hbm, i_hbm, oi_hbm, ou_hbm, sems):
    ins = (i_hbm, u_hbm)
    outs = (oi_hbm, ou_hbm)

    def cp(k):
        t, c = k % 2, k // 2
        return pltpu.make_async_copy(
            ins[t].at[pl.ds(c * _CH, _CH), :],
            outs[t].at[pl.ds(c * _CH, _CH), :],
            sems.at[k],
        )

    for k in range(_TOT):
        cp(k).start()
    for k in range(_TOT):
        cp(k).wait()


def kernel(embed_user, embed_item):
    out_shape = (
        jax.ShapeDtypeStruct(embed_item.shape, embed_item.dtype),
        jax.ShapeDtypeStruct(embed_user.shape, embed_user.dtype),
    )
    return pl.pallas_call(
        _copy_body,
        out_shape=out_shape,
        in_specs=[
            pl.BlockSpec(memory_space=pl.ANY),
            pl.BlockSpec(memory_space=pl.ANY),
        ],
        out_specs=(
            pl.BlockSpec(memory_space=pl.ANY),
            pl.BlockSpec(memory_space=pl.ANY),
        ),
        scratch_shapes=[
            pltpu.SemaphoreType.DMA((_TOT,)),
        ],
    )(embed_user, embed_item)


# speedup vs baseline: 11.5768x; 11.5768x over previous
import jax
import jax.numpy as jnp
from jax import lax
from jax.experimental import pallas as pl
from jax.experimental.pallas import tpu as pltpu
from jax.experimental.pallas import tpu_sc as plsc

_NROW = 100000
_D = 64
_N = _NROW * _D
_NC = 2
_NS = 16
_NW = _NC * _NS
_CH = 50000
_CPT = _N // _CH
_KPT = _CPT // _NW
_K = 2 * _KPT
_NBUF = 2


def _sc_body(u_hbm, i_hbm, oi_hbm, ou_hbm, buf0, buf1, in_sems, out_sems):
    bufs = (buf0, buf1)
    wid = lax.axis_index("s") * _NC + lax.axis_index("c")

    def loc(k):
        t = k // _KPT
        j = k % _KPT
        off = (j * _NW + wid) * _CH
        return t, off

    def in_cp(k, slot):
        t, off = loc(k)
        src = (i_hbm, u_hbm)[t]
        return pltpu.make_async_copy(
            src.at[pl.ds(off, _CH)], bufs[slot], in_sems.at[slot]
        )

    def out_cp(k, slot):
        t, off = loc(k)
        dst = (oi_hbm, ou_hbm)[t]
        return pltpu.make_async_copy(
            bufs[slot], dst.at[pl.ds(off, _CH)], out_sems.at[slot]
        )

    in_cp(0, 0).start()
    for k in range(_K):
        s = k % _NBUF
        nk = k + 1
        if nk < _K:
            ns = nk % _NBUF
            if nk >= _NBUF:
                out_cp(nk - _NBUF, ns).wait()
            in_cp(nk, ns).start()
        in_cp(k, s).wait()
        out_cp(k, s).start()
    for k in range(_K - _NBUF, _K):
        out_cp(k, k % _NBUF).wait()


def kernel(embed_user, embed_item):
    u = embed_user.reshape(_N)
    it = embed_item.reshape(_N)
    out_type = (
        jax.ShapeDtypeStruct((_N,), embed_item.dtype),
        jax.ShapeDtypeStruct((_N,), embed_user.dtype),
    )
    f = pl.kernel(
        _sc_body,
        out_type=out_type,
        mesh=plsc.VectorSubcoreMesh(core_axis_name="c", subcore_axis_name="s"),
        scratch_types=[
            pltpu.VMEM((_CH,), jnp.float32),
            pltpu.VMEM((_CH,), jnp.float32),
            pltpu.SemaphoreType.DMA((_NBUF,)),
            pltpu.SemaphoreType.DMA((_NBUF,)),
        ],
    )
    oi, ou = f(u, it)
    return (
        oi.reshape(_NROW, _D),
        ou.reshape(_NROW, _D),
    )


# speedup vs baseline: 14.9215x; 1.2889x over previous
import jax
import jax.numpy as jnp
from jax import lax
from jax.experimental import pallas as pl
from jax.experimental.pallas import tpu as pltpu
from jax.experimental.pallas import tpu_sc as plsc

_NROW = 100000
_D = 64

_NC = 2
_NS = 16
_NW = _NC * _NS
_CH = 400
_CPT = _NROW // _CH
_KPT = 8
_NBUF = 2
_EXTRA = _CPT - (_KPT - 1) * _NW


def _sc_body(i_hbm, oi_hbm, bufs, in_sems, out_sems):
    wid = lax.axis_index("s") * _NC + lax.axis_index("c")
    has_extra = wid < _EXTRA

    def loc(k):
        r = (k * _NW + wid) * _CH
        return jnp.minimum(r, _NROW - _CH)

    def static_valid(k):
        return k != (_KPT - 1)

    def in_cp(k, slot):
        return pltpu.make_async_copy(
            i_hbm.at[pl.ds(loc(k), _CH), :], bufs.at[slot], in_sems.at[slot]
        )

    def out_cp(k, slot):
        return pltpu.make_async_copy(
            bufs.at[slot], oi_hbm.at[pl.ds(loc(k), _CH), :], out_sems.at[slot]
        )

    def guarded(k, fn):
        if static_valid(k):
            fn()
        else:
            @pl.when(has_extra)
            def _():
                fn()

    in_cp(0, 0).start()
    for k in range(_KPT):
        s = k % _NBUF
        nk = k + 1
        if nk < _KPT:
            ns = nk % _NBUF
            if nk >= _NBUF:
                guarded(nk - _NBUF, out_cp(nk - _NBUF, ns).wait)
            guarded(nk, in_cp(nk, ns).start)
        guarded(k, in_cp(k, s).wait)
        guarded(k, out_cp(k, s).start)
    for k in range(_KPT - _NBUF, _KPT):
        guarded(k, out_cp(k, k % _NBUF).wait)


def _sc_copy(embed_item):
    f = pl.kernel(
        _sc_body,
        out_type=jax.ShapeDtypeStruct(embed_item.shape, embed_item.dtype),
        mesh=plsc.VectorSubcoreMesh(core_axis_name="c", subcore_axis_name="s"),
        scratch_types=[
            pltpu.VMEM((_NBUF, _CH, _D), jnp.float32),
            pltpu.SemaphoreType.DMA((_NBUF,)),
            pltpu.SemaphoreType.DMA((_NBUF,)),
        ],
    )
    return f(embed_item)


_TCH = 2000
_TNC = _NROW // _TCH
_TNBUF = 8
_TH = 4


def _tc_body(u_hbm, ou_hbm, bufs, in_sems, out_sems):
    def in_cp(k):
        return pltpu.make_async_copy(
            u_hbm.at[pl.ds(k * _TCH, _TCH), :],
            bufs.at[k % _TNBUF],
            in_sems.at[k % _TNBUF],
        )

    def out_cp(k):
        return pltpu.make_async_copy(
            bufs.at[k % _TNBUF],
            ou_hbm.at[pl.ds(k * _TCH, _TCH), :],
            out_sems.at[k % _TNBUF],
        )

    for k in range(_TH):
        in_cp(k).start()
    outs_waited = 0
    for k in range(_TNC):
        in_cp(k).wait()
        out_cp(k).start()
        nk = k + _TH
        if nk < _TNC:
            old = nk - _TNBUF
            if old >= 0:
                out_cp(old).wait()
                outs_waited = old + 1
            in_cp(nk).start()
    for k in range(outs_waited, _TNC):
        out_cp(k).wait()


def _tc_copy(embed_user):
    return pl.pallas_call(
        _tc_body,
        out_shape=jax.ShapeDtypeStruct(embed_user.shape, embed_user.dtype),
        in_specs=[pl.BlockSpec(memory_space=pl.ANY)],
        out_specs=pl.BlockSpec(memory_space=pl.ANY),
        scratch_shapes=[
            pltpu.VMEM((_TNBUF, _TCH, _D), jnp.float32),
            pltpu.SemaphoreType.DMA((_TNBUF,)),
            pltpu.SemaphoreType.DMA((_TNBUF,)),
        ],
    )(embed_user)


def kernel(embed_user, embed_item):
    out_item = _sc_copy(embed_item)
    out_user = _tc_copy(embed_user)
    return out_item, out_user
